# Initial kernel scaffold; baseline (speedup 1.0000x reference)
import jax
import jax.numpy as jnp
from jax.experimental import pallas as pl


def kernel(x, edge_index, W1, b1, W2, b2):
    raise NotImplementedError("write your pallas kernel here")


# speedup vs baseline: 9.4000x; 9.4000x over previous
import functools

import jax
import jax.numpy as jnp
from jax import lax
from jax.experimental import pallas as pl
from jax.experimental.pallas import tpu as pltpu
from jax.experimental.pallas import tpu_sc as plsc

N_NODES = 10000
N_EDGES = 320000
C = 128

NC, NS = 2, 16
NW = NC * NS
CHUNK = 128
CPW = 80
E_PAD = NW * CPW * CHUNK
N_PAD = 10240
ROWS_PER_SUB = N_PAD // NS
DUMMY = N_NODES
BLK = 512


def _sc_mesh():
    return plsc.VectorSubcoreMesh(core_axis_name="c", subcore_axis_name="s")


def _sc_degree(dst2d, ones_rows, zeros128):
    @functools.partial(
        pl.kernel,
        mesh=_sc_mesh(),
        out_type=jax.ShapeDtypeStruct((NC * N_PAD, C), jnp.float32),
        scratch_types=[
            pltpu.VMEM((CPW, CHUNK), jnp.int32),
            pltpu.VMEM((CHUNK, C), jnp.float32),
            pltpu.VMEM_SHARED((N_PAD, C), jnp.float32),
        ],
    )
    def deg_kernel(dst_hbm, ones_hbm, zeros_hbm, out_hbm, idx_v, ones_v, acc):
        c = lax.axis_index("c")
        s = lax.axis_index("s")
        wid = s * NC + c
        pltpu.sync_copy(zeros_hbm, acc.at[pl.ds(s * ROWS_PER_SUB, ROWS_PER_SUB)])
        pltpu.sync_copy(ones_hbm, ones_v)
        pltpu.sync_copy(dst_hbm.at[pl.ds(wid * CPW, CPW)], idx_v)
        plsc.subcore_barrier()

        @pl.loop(0, CPW)
        def _(j):
            pltpu.sync_copy(ones_v, acc.at[idx_v.at[j]], add=True)

        plsc.subcore_barrier()
        pltpu.sync_copy(
            acc.at[pl.ds(s * ROWS_PER_SUB, ROWS_PER_SUB)],
            out_hbm.at[pl.ds(c * N_PAD + s * ROWS_PER_SUB, ROWS_PER_SUB)],
        )

    return deg_kernel(dst2d, ones_rows, zeros128)


def _sc_aggregate(y_pad, src2d, dst2d, zeros128):
    @functools.partial(
        pl.kernel,
        mesh=_sc_mesh(),
        out_type=jax.ShapeDtypeStruct((NC * N_PAD, C), jnp.float32),
        scratch_types=[
            pltpu.VMEM((CPW, CHUNK), jnp.int32),
            pltpu.VMEM((CPW, CHUNK), jnp.int32),
            pltpu.VMEM((CHUNK, C), jnp.float32),
            pltpu.VMEM_SHARED((N_PAD, C), jnp.float32),
            pltpu.SemaphoreType.DMA,
        ],
    )
    def agg_kernel(y_hbm, src_hbm, dst_hbm, zeros_hbm, out_hbm,
                   srcv, dstv, rows, acc, sem):
        c = lax.axis_index("c")
        s = lax.axis_index("s")
        wid = s * NC + c
        pltpu.sync_copy(zeros_hbm, acc.at[pl.ds(s * ROWS_PER_SUB, ROWS_PER_SUB)])
        pltpu.sync_copy(src_hbm.at[pl.ds(wid * CPW, CPW)], srcv)
        pltpu.sync_copy(dst_hbm.at[pl.ds(wid * CPW, CPW)], dstv)
        plsc.subcore_barrier()

        @pl.loop(0, CPW)
        def _(j):
            pltpu.async_copy(y_hbm.at[srcv.at[j]], rows, sem).wait()
            pltpu.sync_copy(rows, acc.at[dstv.at[j]], add=True)

        plsc.subcore_barrier()
        pltpu.sync_copy(
            acc.at[pl.ds(s * ROWS_PER_SUB, ROWS_PER_SUB)],
            out_hbm.at[pl.ds(c * N_PAD + s * ROWS_PER_SUB, ROWS_PER_SUB)],
        )

    return agg_kernel(y_pad, src2d, dst2d, zeros128)


def _dinv_from_parts(d_ref):
    cnt = d_ref[0, :, 0:1] + d_ref[1, :, 0:1]
    return lax.rsqrt(cnt + 1.0)


def _tc_matmul1(x_pad, W1):
    def body(x_ref, w_ref, xw_ref):
        xw_ref[...] = jnp.dot(x_ref[...], w_ref[...],
                              preferred_element_type=jnp.float32)

    return pl.pallas_call(
        body,
        grid=(N_PAD // BLK,),
        in_specs=[
            pl.BlockSpec((BLK, C), lambda i: (i, 0)),
            pl.BlockSpec((C, C), lambda i: (0, 0)),
        ],
        out_specs=pl.BlockSpec((BLK, C), lambda i: (i, 0)),
        out_shape=jax.ShapeDtypeStruct((N_PAD, C), jnp.float32),
    )(x_pad, W1)


def _tc_scale1(xw1, degp):
    def body(xw_ref, d_ref, y_ref):
        y_ref[...] = xw_ref[...] * _dinv_from_parts(d_ref)

    return pl.pallas_call(
        body,
        grid=(N_PAD // BLK,),
        in_specs=[
            pl.BlockSpec((BLK, C), lambda i: (i, 0)),
            pl.BlockSpec((NC, BLK, C), lambda i: (0, i, 0)),
        ],
        out_specs=pl.BlockSpec((BLK, C), lambda i: (i, 0)),
        out_shape=jax.ShapeDtypeStruct((N_PAD, C), jnp.float32),
    )(xw1, degp)


def _tc_layer2(zp, xw1, b1, W2, degp):
    def body(z_ref, xw_ref, b_ref, w_ref, d_ref, y_ref, xw2_ref):
        dinv = _dinv_from_parts(d_ref)
        z = z_ref[0] + z_ref[1]
        h = dinv * z + (dinv * dinv) * xw_ref[...] + b_ref[...]
        h = jnp.maximum(h, 0.0)
        xw2 = jnp.dot(h, w_ref[...], preferred_element_type=jnp.float32)
        xw2_ref[...] = xw2
        y_ref[...] = xw2 * dinv

    return pl.pallas_call(
        body,
        grid=(N_PAD // BLK,),
        in_specs=[
            pl.BlockSpec((NC, BLK, C), lambda i: (0, i, 0)),
            pl.BlockSpec((BLK, C), lambda i: (i, 0)),
            pl.BlockSpec((1, C), lambda i: (0, 0)),
            pl.BlockSpec((C, C), lambda i: (0, 0)),
            pl.BlockSpec((NC, BLK, C), lambda i: (0, i, 0)),
        ],
        out_specs=[
            pl.BlockSpec((BLK, C), lambda i: (i, 0)),
            pl.BlockSpec((BLK, C), lambda i: (i, 0)),
        ],
        out_shape=[jax.ShapeDtypeStruct((N_PAD, C), jnp.float32)] * 2,
    )(zp, xw1, b1, W2, degp)


def _tc_final(zp, xw2, b2, degp):
    def body(z_ref, xw_ref, b_ref, d_ref, o_ref):
        dinv = _dinv_from_parts(d_ref)
        z = z_ref[0] + z_ref[1]
        o_ref[...] = dinv * z + (dinv * dinv) * xw_ref[...] + b_ref[...]

    return pl.pallas_call(
        body,
        grid=(N_PAD // BLK,),
        in_specs=[
            pl.BlockSpec((NC, BLK, C), lambda i: (0, i, 0)),
            pl.BlockSpec((BLK, C), lambda i: (i, 0)),
            pl.BlockSpec((1, C), lambda i: (0, 0)),
            pl.BlockSpec((NC, BLK, C), lambda i: (0, i, 0)),
        ],
        out_specs=pl.BlockSpec((BLK, C), lambda i: (i, 0)),
        out_shape=jax.ShapeDtypeStruct((N_PAD, C), jnp.float32),
    )(zp, xw2, b2, degp)


def kernel(x, edge_index, W1, b1, W2, b2):
    ei = edge_index.astype(jnp.int32)
    pad = jnp.full((E_PAD - N_EDGES,), DUMMY, dtype=jnp.int32)
    src2d = jnp.concatenate([ei[0], pad]).reshape(E_PAD // CHUNK, CHUNK)
    dst2d = jnp.concatenate([ei[1], pad]).reshape(E_PAD // CHUNK, CHUNK)

    x_pad = jnp.zeros((N_PAD, C), jnp.float32).at[:N_NODES].set(x)
    ones_rows = jnp.ones((CHUNK, C), jnp.float32)
    zeros128 = jnp.zeros((ROWS_PER_SUB, C), jnp.float32)
    b1r = b1.reshape(1, C)
    b2r = b2.reshape(1, C)

    degp = _sc_degree(dst2d, ones_rows, zeros128).reshape(NC, N_PAD, C)
    xw1 = _tc_matmul1(x_pad, W1)
    y1 = _tc_scale1(xw1, degp)
    z1 = _sc_aggregate(y1, src2d, dst2d, zeros128).reshape(NC, N_PAD, C)
    y2, xw2 = _tc_layer2(z1, xw1, b1r, W2, degp)
    z2 = _sc_aggregate(y2, src2d, dst2d, zeros128).reshape(NC, N_PAD, C)
    out = _tc_final(z2, xw2, b2r, degp)
    return out[:N_NODES]


# speedup vs baseline: 10.7327x; 1.1418x over previous
import functools

import jax
import jax.numpy as jnp
from jax import lax
from jax.experimental import pallas as pl
from jax.experimental.pallas import tpu as pltpu
from jax.experimental.pallas import tpu_sc as plsc

N_NODES = 10000
N_EDGES = 320000
C = 128

NC, NS = 2, 16
NW = NC * NS
CHUNK = 128
CPW = 80
E_PAD = NW * CPW * CHUNK
N_PAD = 10240
ROWS_PER_SUB = N_PAD // NS
DUMMY = N_NODES
BLK = 512
NBUF = 2
HCPW = CPW // 2


def _sc_mesh():
    return plsc.VectorSubcoreMesh(core_axis_name="c", subcore_axis_name="s")


def _sc_degree(dst2d, ones_rows, zeros128):
    @functools.partial(
        pl.kernel,
        mesh=_sc_mesh(),
        out_type=jax.ShapeDtypeStruct((NC * N_PAD, C), jnp.float32),
        scratch_types=[
            pltpu.VMEM((CPW, CHUNK), jnp.int32),
            pltpu.VMEM((CHUNK, C), jnp.float32),
            pltpu.VMEM_SHARED((N_PAD, C), jnp.float32),
        ],
    )
    def deg_kernel(dst_hbm, ones_hbm, zeros_hbm, out_hbm, idx_v, ones_v, acc):
        c = lax.axis_index("c")
        s = lax.axis_index("s")
        wid = s * NC + c
        pltpu.sync_copy(zeros_hbm, acc.at[pl.ds(s * ROWS_PER_SUB, ROWS_PER_SUB)])
        pltpu.sync_copy(ones_hbm, ones_v)
        pltpu.sync_copy(dst_hbm.at[pl.ds(wid * CPW, CPW)], idx_v)
        plsc.subcore_barrier()

        @pl.loop(0, CPW)
        def _(j):
            pltpu.sync_copy(ones_v, acc.at[idx_v.at[j]], add=True)

        plsc.subcore_barrier()
        pltpu.sync_copy(
            acc.at[pl.ds(s * ROWS_PER_SUB, ROWS_PER_SUB)],
            out_hbm.at[pl.ds(c * N_PAD + s * ROWS_PER_SUB, ROWS_PER_SUB)],
        )

    return deg_kernel(dst2d, ones_rows, zeros128)


def _sc_aggregate(y_pad, src2d, dst2d, zeros128):
    @functools.partial(
        pl.kernel,
        mesh=_sc_mesh(),
        out_type=jax.ShapeDtypeStruct((NC * N_PAD, C), jnp.float32),
        scratch_types=[
            pltpu.VMEM((HCPW, CHUNK), jnp.int32),
            pltpu.VMEM((HCPW, CHUNK), jnp.int32),
            pltpu.VMEM((CHUNK, C), jnp.float32),
            pltpu.VMEM((CHUNK, C), jnp.float32),
            pltpu.VMEM_SHARED((N_PAD, C), jnp.float32),
            pltpu.SemaphoreType.DMA,
            pltpu.SemaphoreType.DMA,
        ],
    )
    def agg_kernel(y_hbm, src_hbm, dst_hbm, zeros_hbm, out_hbm,
                   srcv, dstv, rows0, rows1, acc, sem0, sem1):
        c = lax.axis_index("c")
        s = lax.axis_index("s")
        wid = s * NC + c
        rows = (rows0, rows1)
        sems = (sem0, sem1)
        pltpu.sync_copy(zeros_hbm, acc.at[pl.ds(s * ROWS_PER_SUB, ROWS_PER_SUB)])
        plsc.subcore_barrier()

        for h in range(CPW // HCPW):
            pltpu.sync_copy(
                src_hbm.at[pl.ds(wid * CPW + h * HCPW, HCPW)], srcv)
            pltpu.sync_copy(
                dst_hbm.at[pl.ds(wid * CPW + h * HCPW, HCPW)], dstv)

            for b in range(NBUF):
                pltpu.async_copy(y_hbm.at[srcv.at[b]], rows[b], sems[b])

            @pl.loop(0, HCPW, step=NBUF)
            def _(j):
                for b in range(NBUF):
                    pltpu.make_async_copy(
                        y_hbm.at[srcv.at[j + b]], rows[b], sems[b]).wait()
                    pltpu.sync_copy(rows[b], acc.at[dstv.at[j + b]], add=True)

                    @pl.when(j + b + NBUF < HCPW)
                    def _(b=b):
                        pltpu.async_copy(
                            y_hbm.at[srcv.at[j + b + NBUF]], rows[b], sems[b])

        plsc.subcore_barrier()
        pltpu.sync_copy(
            acc.at[pl.ds(s * ROWS_PER_SUB, ROWS_PER_SUB)],
            out_hbm.at[pl.ds(c * N_PAD + s * ROWS_PER_SUB, ROWS_PER_SUB)],
        )

    return agg_kernel(y_pad, src2d, dst2d, zeros128)


def _dinv_from_parts(d_ref):
    cnt = d_ref[0, :, 0:1] + d_ref[1, :, 0:1]
    return lax.rsqrt(cnt + 1.0)


def _tc_matmul1(x_pad, W1):
    def body(x_ref, w_ref, xw_ref):
        xw_ref[...] = jnp.dot(x_ref[...], w_ref[...],
                              preferred_element_type=jnp.float32)

    return pl.pallas_call(
        body,
        grid=(N_PAD // BLK,),
        in_specs=[
            pl.BlockSpec((BLK, C), lambda i: (i, 0)),
            pl.BlockSpec((C, C), lambda i: (0, 0)),
        ],
        out_specs=pl.BlockSpec((BLK, C), lambda i: (i, 0)),
        out_shape=jax.ShapeDtypeStruct((N_PAD, C), jnp.float32),
    )(x_pad, W1)


def _tc_scale1(xw1, degp):
    def body(xw_ref, d_ref, y_ref):
        y_ref[...] = xw_ref[...] * _dinv_from_parts(d_ref)

    return pl.pallas_call(
        body,
        grid=(N_PAD // BLK,),
        in_specs=[
            pl.BlockSpec((BLK, C), lambda i: (i, 0)),
            pl.BlockSpec((NC, BLK, C), lambda i: (0, i, 0)),
        ],
        out_specs=pl.BlockSpec((BLK, C), lambda i: (i, 0)),
        out_shape=jax.ShapeDtypeStruct((N_PAD, C), jnp.float32),
    )(xw1, degp)


def _tc_layer2(zp, xw1, b1, W2, degp):
    def body(z_ref, xw_ref, b_ref, w_ref, d_ref, y_ref, xw2_ref):
        dinv = _dinv_from_parts(d_ref)
        z = z_ref[0] + z_ref[1]
        h = dinv * z + (dinv * dinv) * xw_ref[...] + b_ref[...]
        h = jnp.maximum(h, 0.0)
        xw2 = jnp.dot(h, w_ref[...], preferred_element_type=jnp.float32)
        xw2_ref[...] = xw2
        y_ref[...] = xw2 * dinv

    return pl.pallas_call(
        body,
        grid=(N_PAD // BLK,),
        in_specs=[
            pl.BlockSpec((NC, BLK, C), lambda i: (0, i, 0)),
            pl.BlockSpec((BLK, C), lambda i: (i, 0)),
            pl.BlockSpec((1, C), lambda i: (0, 0)),
            pl.BlockSpec((C, C), lambda i: (0, 0)),
            pl.BlockSpec((NC, BLK, C), lambda i: (0, i, 0)),
        ],
        out_specs=[
            pl.BlockSpec((BLK, C), lambda i: (i, 0)),
            pl.BlockSpec((BLK, C), lambda i: (i, 0)),
        ],
        out_shape=[jax.ShapeDtypeStruct((N_PAD, C), jnp.float32)] * 2,
    )(zp, xw1, b1, W2, degp)


def _tc_final(zp, xw2, b2, degp):
    def body(z_ref, xw_ref, b_ref, d_ref, o_ref):
        dinv = _dinv_from_parts(d_ref)
        z = z_ref[0] + z_ref[1]
        o_ref[...] = dinv * z + (dinv * dinv) * xw_ref[...] + b_ref[...]

    return pl.pallas_call(
        body,
        grid=(N_PAD // BLK,),
        in_specs=[
            pl.BlockSpec((NC, BLK, C), lambda i: (0, i, 0)),
            pl.BlockSpec((BLK, C), lambda i: (i, 0)),
            pl.BlockSpec((1, C), lambda i: (0, 0)),
            pl.BlockSpec((NC, BLK, C), lambda i: (0, i, 0)),
        ],
        out_specs=pl.BlockSpec((BLK, C), lambda i: (i, 0)),
        out_shape=jax.ShapeDtypeStruct((N_PAD, C), jnp.float32),
    )(zp, xw2, b2, degp)


def kernel(x, edge_index, W1, b1, W2, b2):
    ei = edge_index.astype(jnp.int32)
    pad = jnp.full((E_PAD - N_EDGES,), DUMMY, dtype=jnp.int32)
    src2d = jnp.concatenate([ei[0], pad]).reshape(E_PAD // CHUNK, CHUNK)
    dst2d = jnp.concatenate([ei[1], pad]).reshape(E_PAD // CHUNK, CHUNK)

    x_pad = jnp.zeros((N_PAD, C), jnp.float32).at[:N_NODES].set(x)
    ones_rows = jnp.ones((CHUNK, C), jnp.float32)
    zeros128 = jnp.zeros((ROWS_PER_SUB, C), jnp.float32)
    b1r = b1.reshape(1, C)
    b2r = b2.reshape(1, C)

    degp = _sc_degree(dst2d, ones_rows, zeros128).reshape(NC, N_PAD, C)
    xw1 = _tc_matmul1(x_pad, W1)
    y1 = _tc_scale1(xw1, degp)
    z1 = _sc_aggregate(y1, src2d, dst2d, zeros128).reshape(NC, N_PAD, C)
    y2, xw2 = _tc_layer2(z1, xw1, b1r, W2, degp)
    z2 = _sc_aggregate(y2, src2d, dst2d, zeros128).reshape(NC, N_PAD, C)
    out = _tc_final(z2, xw2, b2r, degp)
    return out[:N_NODES]
